# baseline (speedup 1.0000x reference)
import functools

import jax
import jax.numpy as jnp
from jax import lax
from jax.experimental import pallas as pl
from jax.experimental.pallas import tpu as pltpu
from jax.experimental.pallas import tpu_sc as plsc

N = 100000
VAL_DIM = 128
FE = 9
NF = 32

CHUNK = 128
NCHUNK = N // CHUNK
TAIL = N - NCHUNK * CHUNK
CF = CHUNK * FE
VREGS = CF // 16
TROWS = TAIL * FE
NCT = NCHUNK + 1


def _slot_off(k):
    return (k // 4) * 4 * N + (k % 4) if k < 8 else 8 * N


def _proj_body(x_ref, w_ref, o_ref):
    x = x_ref[...]
    for t in range(3):
        o_ref[t] = jnp.dot(x, w_ref[t], preferred_element_type=jnp.float32)


def _project(lattice_values, w3):
    blk = 4000
    return pl.pallas_call(
        _proj_body,
        grid=(N // blk,),
        in_specs=[
            pl.BlockSpec((blk, VAL_DIM), lambda i: (i, 0)),
            pl.BlockSpec((3, VAL_DIM, VAL_DIM), lambda i: (0, 0, 0)),
        ],
        out_specs=pl.BlockSpec((3, blk, VAL_DIM), lambda i: (0, i, 0)),
        out_shape=jax.ShapeDtypeStruct((3, N, VAL_DIM), jnp.float32),
    )(lattice_values, w3)


@functools.partial(
    pl.kernel,
    mesh=plsc.VectorSubcoreMesh(core_axis_name="c", subcore_axis_name="s"),
    compiler_params=pltpu.CompilerParams(use_tc_tiling_on_sc=False,
                                         needs_layout_passes=False),
    out_type=jax.ShapeDtypeStruct((4 * NCT * 8, VAL_DIM), jnp.float32),
    scratch_types=[
        pltpu.VMEM((2, FE, CHUNK), jnp.int32),
        pltpu.VMEM((2, CF), jnp.int32),
        pltpu.VMEM((2, CF, NF), jnp.float32),
        pltpu.VMEM((2, NF, CHUNK), jnp.float32),
        pltpu.VMEM((NF,), jnp.float32),
        pltpu.SemaphoreType.DMA((2,)),
        pltpu.SemaphoreType.DMA((2,)),
    ],
)
def _sc_gather(table, nbr_t, bias, out, idx_v, flat_v, rows_v, acc_v,
               bias_v, sem_g, sem_o):
    cid = lax.axis_index("c")
    sid = lax.axis_index("s")
    wid = sid * 2 + cid
    cbase = 24 * wid + jnp.minimum(wid, 13)
    ncch = jnp.where(wid < 13, 25, 24)

    four_c = jnp.full((16,), 4, dtype=jnp.int32)
    iota16 = lax.iota(jnp.int32, 16)
    fhi = iota16 + jnp.int32(16)
    pltpu.sync_copy(bias, bias_v)
    blo = bias_v[pl.ds(0, 16)]
    bhi = bias_v[pl.ds(16, 16)]

    def wb_copies(b, g, width):
        return [
            pltpu.make_async_copy(
                acc_v.at[b, pl.ds(s * 8, 8), pl.ds(0, width)],
                out.at[pl.ds((s * NCT + g) * 8, 8), pl.ds(0, width)],
                sem_o.at[b])
            for s in range(4)
        ]

    def issue(t):
        b = lax.rem(t, 2)
        g = cbase + t
        pltpu.sync_copy(nbr_t.at[:, pl.ds(g * CHUNK, CHUNK)], idx_v.at[b])
        for v in range(VREGS):
            k = v // 8
            j0 = (v % 8) * 16
            flat_v[b, pl.ds(v * 16, 16)] = (
                idx_v[b, k, pl.ds(j0, 16)] * four_c
                + jnp.full((16,), _slot_off(k), dtype=jnp.int32))
        pltpu.async_copy(table.at[flat_v.at[b]], rows_v.at[b], sem_g.at[b])

    issue(0)

    def chunk_body(t, carry):
        b = lax.rem(t, 2)
        g = cbase + t

        @pl.when(t + 1 < ncch)
        def _():
            issue(t + 1)

        pltpu.make_async_copy(table.at[flat_v.at[b]], rows_v.at[b],
                              sem_g.at[b]).wait()

        @pl.when(t >= 2)
        def _():
            for c in wb_copies(b, g - 2, CHUNK):
                c.wait()

        def vert_body(j, c):
            lo = blo
            hi = bhi
            for k in range(FE):
                lo = lo + rows_v[b, k * CHUNK + j, pl.ds(0, 16)]
                hi = hi + rows_v[b, k * CHUNK + j, pl.ds(16, 16)]
            jc = jnp.full((16,), j, dtype=jnp.int32)
            plsc.store_scatter(acc_v.at[b], [iota16, jc], lo)
            plsc.store_scatter(acc_v.at[b], [fhi, jc], hi)
            return c

        lax.fori_loop(0, CHUNK, vert_body, 0)
        for c in wb_copies(b, g, CHUNK):
            c.start()
        return carry

    lax.fori_loop(0, ncch, chunk_body, 0)

    for dt in (2, 1):
        t = ncch - dt
        b = lax.rem(t, 2)
        for c in wb_copies(b, cbase + t, CHUNK):
            c.wait()

    @pl.when(wid == 31)
    def _():
        pltpu.sync_copy(nbr_t.at[:, pl.ds(NCHUNK * CHUNK, TAIL)],
                        idx_v.at[0, :, pl.ds(0, TAIL)])
        for v in range(TROWS // 16):
            k = v // 2
            j0 = (v % 2) * 16
            flat_v[0, pl.ds(v * 16, 16)] = (
                idx_v[0, k, pl.ds(j0, 16)] * four_c
                + jnp.full((16,), _slot_off(k), dtype=jnp.int32))
        pltpu.async_copy(table.at[flat_v.at[0, pl.ds(0, TROWS)]],
                         rows_v.at[0, pl.ds(0, TROWS)], sem_g.at[0]).wait()

        def tail_body(j, c):
            lo = blo
            hi = bhi
            for k in range(FE):
                lo = lo + rows_v[0, k * TAIL + j, pl.ds(0, 16)]
                hi = hi + rows_v[0, k * TAIL + j, pl.ds(16, 16)]
            jc = jnp.full((16,), j, dtype=jnp.int32)
            plsc.store_scatter(acc_v.at[0], [iota16, jc], lo)
            plsc.store_scatter(acc_v.at[0], [fhi, jc], hi)
            return c

        lax.fori_loop(0, TAIL, tail_body, 0)
        for c in wb_copies(0, NCHUNK, TAIL):
            c.start()
            c.wait()


def kernel(lattice_values, neighbor_indices, weight, bias):
    w_r = weight.reshape(FE, VAL_DIM, NF).transpose(1, 0, 2).reshape(
        VAL_DIM, FE * NF)
    w3 = jnp.pad(w_r, ((0, 0), (0, 3 * VAL_DIM - FE * NF))).reshape(
        VAL_DIM, 3, VAL_DIM).transpose(1, 0, 2)
    proj = _project(lattice_values, w3)
    table = proj.reshape(3 * N * VAL_DIM // NF, NF)
    nbr_t = neighbor_indices.astype(jnp.int32).T
    o4 = _sc_gather(table, nbr_t, bias).reshape(4, NCT, 8, VAL_DIM)
    return o4.transpose(1, 3, 0, 2).reshape(NCT * VAL_DIM, NF)[:N]


# speedup vs baseline: 1.3642x; 1.3642x over previous
import functools

import jax
import jax.numpy as jnp
from jax import lax
from jax.experimental import pallas as pl
from jax.experimental.pallas import tpu as pltpu
from jax.experimental.pallas import tpu_sc as plsc

N = 100000
VAL_DIM = 128
FE = 9
NF = 32

CHUNK = 128
NCHUNK = N // CHUNK
TAIL = N - NCHUNK * CHUNK
CF = CHUNK * FE
VREGS = CF // 16
TROWS = TAIL * FE
NCT = NCHUNK + 1


def _slot_off(k):
    return (k // 4) * 4 * N + (k % 4) if k < 8 else 8 * N


def _proj_body(x_ref, w_ref, o_ref):
    x = x_ref[...]
    for t in range(3):
        o_ref[t] = jnp.dot(x, w_ref[t], preferred_element_type=jnp.float32)


def _project(lattice_values, w3):
    blk = 4000
    return pl.pallas_call(
        _proj_body,
        grid=(N // blk,),
        in_specs=[
            pl.BlockSpec((blk, VAL_DIM), lambda i: (i, 0)),
            pl.BlockSpec((3, VAL_DIM, VAL_DIM), lambda i: (0, 0, 0)),
        ],
        out_specs=pl.BlockSpec((3, blk, VAL_DIM), lambda i: (0, i, 0)),
        out_shape=jax.ShapeDtypeStruct((3, N, VAL_DIM), jnp.float32),
    )(lattice_values, w3)


@functools.partial(
    pl.kernel,
    mesh=plsc.VectorSubcoreMesh(core_axis_name="c", subcore_axis_name="s"),
    compiler_params=pltpu.CompilerParams(use_tc_tiling_on_sc=False,
                                         needs_layout_passes=False),
    out_type=jax.ShapeDtypeStruct((4 * NCT * 8, VAL_DIM), jnp.float32),
    scratch_types=[
        pltpu.VMEM((2, FE, CHUNK), jnp.int32),
        pltpu.VMEM((2, CF), jnp.int32),
        pltpu.VMEM((2, CF, NF), jnp.float32),
        pltpu.VMEM((2, NF, CHUNK + 1), jnp.float32),
        pltpu.VMEM((NF,), jnp.float32),
        pltpu.SemaphoreType.DMA((2,)),
        pltpu.SemaphoreType.DMA((2,)),
    ],
)
def _sc_gather(table, nbr_t, bias, out, idx_v, flat_v, rows_v, acc_v,
               bias_v, sem_g, sem_o):
    cid = lax.axis_index("c")
    sid = lax.axis_index("s")
    wid = sid * 2 + cid
    cbase = 24 * wid + jnp.minimum(wid, 13)
    ncch = jnp.where(wid < 13, 25, 24)

    four_c = jnp.full((16,), 4, dtype=jnp.int32)
    iota16 = lax.iota(jnp.int32, 16)
    fhi = iota16 + jnp.int32(16)
    pltpu.sync_copy(bias, bias_v)
    blo = bias_v[pl.ds(0, 16)]
    bhi = bias_v[pl.ds(16, 16)]

    def wb_copies(b, g, width):
        return [
            pltpu.make_async_copy(
                acc_v.at[b, pl.ds(s * 8, 8), pl.ds(0, width)],
                out.at[pl.ds((s * NCT + g) * 8, 8), pl.ds(0, width)],
                sem_o.at[b])
            for s in range(4)
        ]

    def issue(t):
        b = lax.rem(t, 2)
        g = cbase + t
        pltpu.sync_copy(nbr_t.at[:, pl.ds(g * CHUNK, CHUNK)], idx_v.at[b])
        for v in range(VREGS):
            k = v // 8
            j0 = (v % 8) * 16
            flat_v[b, pl.ds(v * 16, 16)] = (
                idx_v[b, k, pl.ds(j0, 16)] * four_c
                + jnp.full((16,), _slot_off(k), dtype=jnp.int32))
        pltpu.async_copy(table.at[flat_v.at[b]], rows_v.at[b], sem_g.at[b])

    issue(0)

    def chunk_body(t, carry):
        b = lax.rem(t, 2)
        g = cbase + t

        @pl.when(t + 1 < ncch)
        def _():
            issue(t + 1)

        pltpu.make_async_copy(table.at[flat_v.at[b]], rows_v.at[b],
                              sem_g.at[b]).wait()

        @pl.when(t >= 2)
        def _():
            for c in wb_copies(b, g - 2, CHUNK):
                c.wait()

        def vert_body(j, c):
            lo = blo
            hi = bhi
            for k in range(FE):
                lo = lo + rows_v[b, k * CHUNK + j, pl.ds(0, 16)]
                hi = hi + rows_v[b, k * CHUNK + j, pl.ds(16, 16)]
            jc = jnp.full((16,), j, dtype=jnp.int32)
            plsc.store_scatter(acc_v.at[b], [iota16, jc], lo)
            plsc.store_scatter(acc_v.at[b], [fhi, jc], hi)
            return c

        lax.fori_loop(0, CHUNK, vert_body, 0)
        for c in wb_copies(b, g, CHUNK):
            c.start()
        return carry

    lax.fori_loop(0, ncch, chunk_body, 0)

    for dt in (2, 1):
        t = ncch - dt
        b = lax.rem(t, 2)
        for c in wb_copies(b, cbase + t, CHUNK):
            c.wait()

    @pl.when(wid == 31)
    def _():
        pltpu.sync_copy(nbr_t.at[:, pl.ds(NCHUNK * CHUNK, TAIL)],
                        idx_v.at[0, :, pl.ds(0, TAIL)])
        for v in range(TROWS // 16):
            k = v // 2
            j0 = (v % 2) * 16
            flat_v[0, pl.ds(v * 16, 16)] = (
                idx_v[0, k, pl.ds(j0, 16)] * four_c
                + jnp.full((16,), _slot_off(k), dtype=jnp.int32))
        pltpu.async_copy(table.at[flat_v.at[0, pl.ds(0, TROWS)]],
                         rows_v.at[0, pl.ds(0, TROWS)], sem_g.at[0]).wait()

        def tail_body(j, c):
            lo = blo
            hi = bhi
            for k in range(FE):
                lo = lo + rows_v[0, k * TAIL + j, pl.ds(0, 16)]
                hi = hi + rows_v[0, k * TAIL + j, pl.ds(16, 16)]
            jc = jnp.full((16,), j, dtype=jnp.int32)
            plsc.store_scatter(acc_v.at[0], [iota16, jc], lo)
            plsc.store_scatter(acc_v.at[0], [fhi, jc], hi)
            return c

        lax.fori_loop(0, TAIL, tail_body, 0)
        for c in wb_copies(0, NCHUNK, TAIL):
            c.start()
            c.wait()


def kernel(lattice_values, neighbor_indices, weight, bias):
    w_r = weight.reshape(FE, VAL_DIM, NF).transpose(1, 0, 2).reshape(
        VAL_DIM, FE * NF)
    w3 = jnp.pad(w_r, ((0, 0), (0, 3 * VAL_DIM - FE * NF))).reshape(
        VAL_DIM, 3, VAL_DIM).transpose(1, 0, 2)
    proj = _project(lattice_values, w3)
    table = proj.reshape(3 * N * VAL_DIM // NF, NF)
    nbr_t = neighbor_indices.astype(jnp.int32).T
    o4 = _sc_gather(table, nbr_t, bias).reshape(4, NCT, 8, VAL_DIM)
    return o4.transpose(1, 3, 0, 2).reshape(NCT * VAL_DIM, NF)[:N]
